# Initial kernel scaffold; baseline (speedup 1.0000x reference)
import jax
import jax.numpy as jnp
from jax.experimental import pallas as pl


def kernel(input, adj, W, bn_weight, bn_bias):
    raise NotImplementedError("write your pallas kernel here")


# speedup vs baseline: 1.0328x; 1.0328x over previous
import jax
import jax.numpy as jnp
from jax.experimental import pallas as pl

_N = 10000
_D = 128
_BM = 400
_BR = 2000
_BN_EPS = 1e-5


def _mm_kernel(adj_ref, x_ref, w_ref, out_ref, cs_ref, css_ref):
    i = pl.program_id(0)
    tmp = jnp.dot(adj_ref[...], x_ref[...], preferred_element_type=jnp.float32)
    out = jnp.dot(tmp, w_ref[...], preferred_element_type=jnp.float32)
    out_ref[...] = out
    s = jnp.sum(out, axis=0, keepdims=True)
    sq = jnp.sum(out * out, axis=0, keepdims=True)

    @pl.when(i == 0)
    def _():
        cs_ref[...] = s
        css_ref[...] = sq

    @pl.when(i != 0)
    def _():
        cs_ref[...] = cs_ref[...] + s
        css_ref[...] = css_ref[...] + sq


def _bn_kernel(out_ref, cs_ref, css_ref, g_ref, b_ref, y_ref):
    mean = cs_ref[...] * (1.0 / _N)
    var = css_ref[...] * (1.0 / _N) - mean * mean
    inv = jax.lax.rsqrt(var + _BN_EPS)
    y_ref[...] = jnp.tanh((out_ref[...] - mean) * inv * g_ref[...] + b_ref[...])


def kernel(input, adj, W, bn_weight, bn_bias):
    x = input
    out, cs, css = pl.pallas_call(
        _mm_kernel,
        grid=(_N // _BM,),
        in_specs=[
            pl.BlockSpec((_BM, _N), lambda i: (i, 0)),
            pl.BlockSpec((_N, _D), lambda i: (0, 0)),
            pl.BlockSpec((_D, _D), lambda i: (0, 0)),
        ],
        out_specs=[
            pl.BlockSpec((_BM, _D), lambda i: (i, 0)),
            pl.BlockSpec((1, _D), lambda i: (0, 0)),
            pl.BlockSpec((1, _D), lambda i: (0, 0)),
        ],
        out_shape=[
            jax.ShapeDtypeStruct((_N, _D), jnp.float32),
            jax.ShapeDtypeStruct((1, _D), jnp.float32),
            jax.ShapeDtypeStruct((1, _D), jnp.float32),
        ],
    )(adj, x, W)

    g = bn_weight.reshape(1, _D)
    b = bn_bias.reshape(1, _D)
    y = pl.pallas_call(
        _bn_kernel,
        grid=(_N // _BR,),
        in_specs=[
            pl.BlockSpec((_BR, _D), lambda i: (i, 0)),
            pl.BlockSpec((1, _D), lambda i: (0, 0)),
            pl.BlockSpec((1, _D), lambda i: (0, 0)),
            pl.BlockSpec((1, _D), lambda i: (0, 0)),
            pl.BlockSpec((1, _D), lambda i: (0, 0)),
        ],
        out_specs=pl.BlockSpec((_BR, _D), lambda i: (i, 0)),
        out_shape=jax.ShapeDtypeStruct((_N, _D), jnp.float32),
    )(out, cs, css, g, b)
    return y
